# baseline (speedup 1.0000x reference)
import functools
import jax
import jax.numpy as jnp
from jax import lax
from jax.experimental import pallas as pl
from jax.experimental.pallas import tpu as pltpu
from jax.experimental.pallas import tpu_sc as plsc

N = 10000
NF = 128
NC = 10
E = 320000

NCORES = 2
NSUB = 16
NTILES = NCORES * NSUB
CHUNK = 80
CPT = 128
EPAD = NTILES * CPT * CHUNK
ROWS_PER_TILE = 640
PADN = NSUB * ROWS_PER_TILE

_mesh = plsc.VectorSubcoreMesh(
    core_axis_name="c", subcore_axis_name="s",
    num_cores=NCORES, num_subcores=NSUB)


@functools.partial(
    pl.kernel,
    out_type=jax.ShapeDtypeStruct((NCORES * PADN,), jnp.float32),
    mesh=_mesh,
    scratch_types=[
        pltpu.VMEM((CPT, CHUNK), jnp.int32),
        pltpu.VMEM((CPT, CHUNK), jnp.float32),
        pltpu.VMEM((CHUNK,), jnp.float32),
        pltpu.VMEM_SHARED((PADN,), jnp.float32),
        pltpu.SemaphoreType.DMA,
        pltpu.SemaphoreType.DMA,
    ],
)
def _deg_kernel(dst_hbm, ew_hbm, out_hbm, dstall, ewall, zv, deg_sh, semA, semB):
    c = lax.axis_index("c")
    s = lax.axis_index("s")
    zero16 = jnp.zeros((16,), jnp.float32)
    for i in range(CHUNK // 16):
        zv[pl.ds(i * 16, 16)] = zero16
    for i in range(ROWS_PER_TILE // CHUNK):
        pltpu.sync_copy(zv, deg_sh.at[pl.ds(s * ROWS_PER_TILE + i * CHUNK, CHUNK)])
    plsc.subcore_barrier()

    gid = c * NSUB + s
    pltpu.sync_copy(dst_hbm.at[pl.ds(gid * CPT, CPT)], dstall)
    pltpu.sync_copy(ew_hbm.at[pl.ds(gid * CPT, CPT)], ewall)

    def add_chunk(k, sem):
        pltpu.async_copy(ewall.at[k], deg_sh.at[dstall.at[k]], sem, add=True)

    def wait_chunk(k, sem):
        pltpu.make_async_copy(ewall.at[k], deg_sh.at[dstall.at[k]], sem).wait()

    add_chunk(0, semA)
    add_chunk(1, semB)

    def body(t, carry):
        wait_chunk(2 * t, semA)
        add_chunk(2 * t + 2, semA)
        wait_chunk(2 * t + 1, semB)
        add_chunk(2 * t + 3, semB)
        return carry

    lax.fori_loop(0, CPT // 2 - 1, body, 0)
    wait_chunk(CPT - 2, semA)
    wait_chunk(CPT - 1, semB)

    plsc.subcore_barrier()
    pltpu.sync_copy(
        deg_sh.at[pl.ds(s * ROWS_PER_TILE, ROWS_PER_TILE)],
        out_hbm.at[pl.ds(c * PADN + s * ROWS_PER_TILE, ROWS_PER_TILE)])


QCH = 64
NQ = CPT // QCH


@functools.partial(
    pl.kernel,
    out_type=jax.ShapeDtypeStruct((NCORES * PADN, NF), jnp.float32),
    mesh=_mesh,
    scratch_types=[
        pltpu.VMEM((QCH, CHUNK), jnp.int32),
        pltpu.VMEM((QCH, CHUNK), jnp.int32),
        pltpu.VMEM((QCH, CHUNK), jnp.float32),
        pltpu.VMEM((CHUNK, NF), jnp.float32),
        pltpu.VMEM((CHUNK, NF), jnp.float32),
        pltpu.VMEM_SHARED((PADN, NF), jnp.float32),
        pltpu.SemaphoreType.DMA,
        pltpu.SemaphoreType.DMA,
        pltpu.SemaphoreType.DMA,
        pltpu.SemaphoreType.DMA,
    ],
)
def _edge_kernel(y_hbm, src_hbm, dst_hbm, ew_hbm, out_hbm,
                 srcall, dstall, ewall, rows0, rows1, acc_sh,
                 g0, g1, s0, s1):
    c = lax.axis_index("c")
    s = lax.axis_index("s")
    zero16 = jnp.zeros((16,), jnp.float32)
    for e in range(CHUNK):
        for j in range(NF // 16):
            rows1[e, pl.ds(j * 16, 16)] = zero16
    for i in range(ROWS_PER_TILE // CHUNK):
        pltpu.sync_copy(rows1, acc_sh.at[pl.ds(s * ROWS_PER_TILE + i * CHUNK, CHUNK)])
    plsc.subcore_barrier()

    gid = c * NSUB + s
    out_base = c * PADN + s * ROWS_PER_TILE

    def gather(k, rows, sem):
        pltpu.async_copy(y_hbm.at[srcall.at[k]], rows, sem)

    def scatter(k, rows, sem):
        pltpu.async_copy(rows, acc_sh.at[dstall.at[k]], sem, add=True)

    def scale(k, rows):
        for b in range(CHUNK // 16):
            wv = ewall[k, pl.ds(b * 16, 16)]
            for l in range(16):
                e = b * 16 + l
                w = wv[l]
                for j in range(NF // 16):
                    rows[e, pl.ds(j * 16, 16)] = rows[e, pl.ds(j * 16, 16)] * w

    def quarter(q, carry):
        base = s * 256 + (c + q) * QCH
        pltpu.sync_copy(src_hbm.at[pl.ds(base, QCH)], srcall)
        pltpu.sync_copy(dst_hbm.at[pl.ds(base, QCH)], dstall)
        pltpu.sync_copy(ew_hbm.at[pl.ds(base, QCH)], ewall)
        pltpu.async_copy(rows1, out_hbm.at[pl.ds(out_base, CHUNK)], s1)
        gather(0, rows0, g0)

        def body(t, cc):
            k = 2 * t
            pltpu.make_async_copy(y_hbm.at[srcall.at[k]], rows0, g0).wait()
            pltpu.make_async_copy(rows1, acc_sh.at[dstall.at[k]], s1).wait()
            gather(k + 1, rows1, g1)
            scale(k, rows0)
            scatter(k, rows0, s0)
            pltpu.make_async_copy(y_hbm.at[srcall.at[k + 1]], rows1, g1).wait()
            pltpu.make_async_copy(rows0, acc_sh.at[dstall.at[k]], s0).wait()
            gather(jnp.minimum(k + 2, QCH - 1), rows0, g0)
            scale(k + 1, rows1)
            scatter(k + 1, rows1, s1)
            return cc

        lax.fori_loop(0, QCH // 2, body, 0)
        pltpu.make_async_copy(y_hbm.at[srcall.at[QCH - 1]], rows0, g0).wait()
        pltpu.make_async_copy(rows1, acc_sh.at[dstall.at[QCH - 1]], s1).wait()
        return carry

    lax.fori_loop(0, 1 + 2 * c, quarter, 0)
    plsc.subcore_barrier()
    pltpu.sync_copy(
        acc_sh.at[pl.ds(s * ROWS_PER_TILE, ROWS_PER_TILE)],
        out_hbm.at[pl.ds(c * PADN + s * ROWS_PER_TILE, ROWS_PER_TILE)])


def _prep_body(numx_ref, cx0_ref, cx1_ref, w0_ref, w1_ref, W_ref,
               d0_ref, d1_ref, y_ref, dinv_ref):
    e0 = lax.dot_general(w0_ref[...], cx0_ref[...],
                         (((1,), (1,)), ((), ())),
                         preferred_element_type=jnp.float32)
    e1 = lax.dot_general(w1_ref[...], cx1_ref[...],
                         (((1,), (1,)), ((), ())),
                         preferred_element_type=jnp.float32)
    x = jnp.concatenate([numx_ref[...], e0, e1], axis=0)
    xw = lax.dot_general(x, W_ref[...],
                         (((1,), (0,)), ((), ())),
                         preferred_element_type=jnp.float32)
    deg = 1.0 + d0_ref[...] + d1_ref[...]
    dinv = lax.rsqrt(deg)
    dinv_ref[...] = dinv
    y_ref[...] = xw * dinv


def _tail_body(acc_ref, y_ref, dinv_ref, van_ref, fcW_ref, fcb_ref, out_ref):
    a = acc_ref[...]
    a0 = a[0:N]
    a1 = a[PADN:PADN + N]
    conv = jnp.maximum((a0 + a1 + y_ref[...]) * dinv_ref[...], 0.0)
    pooled = jnp.sum(conv, axis=0, keepdims=True) * (1.0 / N)
    fcW = fcW_ref[...]
    sc = lax.dot_general(pooled, fcW[NC:NC + NF],
                         (((1,), (0,)), ((), ())),
                         preferred_element_type=jnp.float32)
    z = lax.dot_general(van_ref[...], fcW[0:NC],
                        (((1,), (0,)), ((), ())),
                        preferred_element_type=jnp.float32)
    z = z + sc + fcb_ref[0, 0]
    beta = 1.1
    t = jax.nn.softplus(beta * z) / beta
    out_ref[...] = van_ref[...] / t


def kernel(num_x, cat_x0, cat_x1, edge_index, edge_weight, batch, vanilla_out,
           W_conv, embed_w0, embed_w1, fc_W, fc_b):
    npad = EPAD - E
    src2d = jnp.concatenate(
        [edge_index[0], jnp.zeros((npad,), jnp.int32)]).reshape(EPAD // CHUNK, CHUNK)
    dst2d = jnp.concatenate(
        [edge_index[1], jnp.zeros((npad,), jnp.int32)]).reshape(EPAD // CHUNK, CHUNK)
    ew2d = jnp.concatenate(
        [edge_weight, jnp.zeros((npad,), jnp.float32)]).reshape(EPAD // CHUNK, CHUNK)

    deg_flat = _deg_kernel(dst2d, ew2d)
    d0 = deg_flat[0:N].reshape(N, 1)
    d1 = deg_flat[PADN:PADN + N].reshape(N, 1)

    y, dinv = pl.pallas_call(
        _prep_body,
        out_shape=[
            jax.ShapeDtypeStruct((N, NF), jnp.float32),
            jax.ShapeDtypeStruct((N, 1), jnp.float32),
        ],
    )(num_x, cat_x0, cat_x1,
      embed_w0.reshape(1, 16), embed_w1.reshape(1, 16), W_conv, d0, d1)

    acc = _edge_kernel(y, src2d, dst2d, ew2d)

    out = pl.pallas_call(
        _tail_body,
        out_shape=jax.ShapeDtypeStruct((vanilla_out.shape[0], NC), jnp.float32),
    )(acc, y, dinv, vanilla_out, fc_W, fc_b.reshape(1, 1))
    return out


# speedup vs baseline: 1.3044x; 1.3044x over previous
import functools
import jax
import jax.numpy as jnp
from jax import lax
from jax.experimental import pallas as pl
from jax.experimental.pallas import tpu as pltpu
from jax.experimental.pallas import tpu_sc as plsc

N = 10000
NF = 128
NC = 10
E = 320000

NCORES = 2
NSUB = 16
NTILES = NCORES * NSUB
CHUNK = 80
CPT = 128
EPAD = NTILES * CPT * CHUNK
ROWS_PER_TILE = 640
PADN = NSUB * ROWS_PER_TILE

_mesh = plsc.VectorSubcoreMesh(
    core_axis_name="c", subcore_axis_name="s",
    num_cores=NCORES, num_subcores=NSUB)


@functools.partial(
    pl.kernel,
    out_type=jax.ShapeDtypeStruct((NCORES * PADN,), jnp.float32),
    mesh=_mesh,
    scratch_types=[
        pltpu.VMEM((CPT, CHUNK), jnp.int32),
        pltpu.VMEM((CPT, CHUNK), jnp.float32),
        pltpu.VMEM((CHUNK,), jnp.float32),
        pltpu.VMEM_SHARED((PADN,), jnp.float32),
        pltpu.SemaphoreType.DMA,
        pltpu.SemaphoreType.DMA,
    ],
)
def _deg_kernel(dst_hbm, ew_hbm, out_hbm, dstall, ewall, zv, deg_sh, semA, semB):
    c = lax.axis_index("c")
    s = lax.axis_index("s")
    zero16 = jnp.zeros((16,), jnp.float32)
    for i in range(CHUNK // 16):
        zv[pl.ds(i * 16, 16)] = zero16
    for i in range(ROWS_PER_TILE // CHUNK):
        pltpu.sync_copy(zv, deg_sh.at[pl.ds(s * ROWS_PER_TILE + i * CHUNK, CHUNK)])
    plsc.subcore_barrier()

    gid = c * NSUB + s
    pltpu.sync_copy(dst_hbm.at[pl.ds(gid * CPT, CPT)], dstall)
    pltpu.sync_copy(ew_hbm.at[pl.ds(gid * CPT, CPT)], ewall)

    def add_chunk(k, sem):
        pltpu.async_copy(ewall.at[k], deg_sh.at[dstall.at[k]], sem, add=True)

    def wait_chunk(k, sem):
        pltpu.make_async_copy(ewall.at[k], deg_sh.at[dstall.at[k]], sem).wait()

    add_chunk(0, semA)
    add_chunk(1, semB)

    def body(t, carry):
        wait_chunk(2 * t, semA)
        add_chunk(2 * t + 2, semA)
        wait_chunk(2 * t + 1, semB)
        add_chunk(2 * t + 3, semB)
        return carry

    lax.fori_loop(0, CPT // 2 - 1, body, 0)
    wait_chunk(CPT - 2, semA)
    wait_chunk(CPT - 1, semB)

    plsc.subcore_barrier()
    pltpu.sync_copy(
        deg_sh.at[pl.ds(s * ROWS_PER_TILE, ROWS_PER_TILE)],
        out_hbm.at[pl.ds(c * PADN + s * ROWS_PER_TILE, ROWS_PER_TILE)])


QCH = 64
NQ = CPT // QCH


@functools.partial(
    pl.kernel,
    out_type=jax.ShapeDtypeStruct((NCORES * PADN, NF), jnp.float32),
    mesh=_mesh,
    scratch_types=[
        pltpu.VMEM((QCH, CHUNK), jnp.int32),
        pltpu.VMEM((QCH, CHUNK), jnp.int32),
        pltpu.VMEM((QCH, CHUNK), jnp.float32),
        pltpu.VMEM((CHUNK, NF), jnp.float32),
        pltpu.VMEM((CHUNK, NF), jnp.float32),
        pltpu.VMEM_SHARED((PADN, NF), jnp.float32),
        pltpu.SemaphoreType.DMA,
        pltpu.SemaphoreType.DMA,
        pltpu.SemaphoreType.DMA,
        pltpu.SemaphoreType.DMA,
    ],
)
def _edge_kernel(y_hbm, src_hbm, dst_hbm, ew_hbm, out_hbm,
                 srcall, dstall, ewall, rows0, rows1, acc_sh,
                 g0, g1, s0, s1):
    c = lax.axis_index("c")
    s = lax.axis_index("s")
    zero16 = jnp.zeros((16,), jnp.float32)
    for e in range(CHUNK):
        for j in range(NF // 16):
            rows1[e, pl.ds(j * 16, 16)] = zero16
    for i in range(ROWS_PER_TILE // CHUNK):
        pltpu.sync_copy(rows1, acc_sh.at[pl.ds(s * ROWS_PER_TILE + i * CHUNK, CHUNK)])
    plsc.subcore_barrier()

    gid = c * NSUB + s
    out_base = c * PADN + s * ROWS_PER_TILE

    def gather(k, rows, sem):
        pltpu.async_copy(y_hbm.at[srcall.at[k]], rows, sem)

    def scatter(k, rows, sem):
        pltpu.async_copy(rows, acc_sh.at[dstall.at[k]], sem, add=True)

    def scale(k, rows):
        for b in range(CHUNK // 16):
            wv = ewall[k, pl.ds(b * 16, 16)]
            for l in range(16):
                e = b * 16 + l
                w = wv[l]
                for j in range(NF // 16):
                    rows[e, pl.ds(j * 16, 16)] = rows[e, pl.ds(j * 16, 16)] * w

    def quarter(q, carry):
        base = s * 256 + (3 * c + q) * QCH
        pltpu.sync_copy(src_hbm.at[pl.ds(base, QCH)], srcall)
        pltpu.sync_copy(dst_hbm.at[pl.ds(base, QCH)], dstall)
        pltpu.sync_copy(ew_hbm.at[pl.ds(base, QCH)], ewall)
        pltpu.async_copy(rows1, out_hbm.at[pl.ds(out_base, CHUNK)], s1)
        gather(0, rows0, g0)

        def body(t, cc):
            k = 2 * t
            pltpu.make_async_copy(y_hbm.at[srcall.at[k]], rows0, g0).wait()
            pltpu.make_async_copy(rows1, acc_sh.at[dstall.at[k]], s1).wait()
            gather(k + 1, rows1, g1)
            scale(k, rows0)
            scatter(k, rows0, s0)
            pltpu.make_async_copy(y_hbm.at[srcall.at[k + 1]], rows1, g1).wait()
            pltpu.make_async_copy(rows0, acc_sh.at[dstall.at[k]], s0).wait()
            gather(jnp.minimum(k + 2, QCH - 1), rows0, g0)
            scale(k + 1, rows1)
            scatter(k + 1, rows1, s1)
            return cc

        lax.fori_loop(0, QCH // 2, body, 0)
        pltpu.make_async_copy(y_hbm.at[srcall.at[QCH - 1]], rows0, g0).wait()
        pltpu.make_async_copy(rows1, acc_sh.at[dstall.at[QCH - 1]], s1).wait()
        return carry

    lax.fori_loop(0, 3 - 2 * c, quarter, 0)
    plsc.subcore_barrier()
    pltpu.sync_copy(
        acc_sh.at[pl.ds(s * ROWS_PER_TILE, ROWS_PER_TILE)],
        out_hbm.at[pl.ds(c * PADN + s * ROWS_PER_TILE, ROWS_PER_TILE)])


def _prep_body(numx_ref, cx0_ref, cx1_ref, w0_ref, w1_ref, W_ref,
               d0_ref, d1_ref, y_ref, dinv_ref):
    e0 = lax.dot_general(w0_ref[...], cx0_ref[...],
                         (((1,), (1,)), ((), ())),
                         preferred_element_type=jnp.float32)
    e1 = lax.dot_general(w1_ref[...], cx1_ref[...],
                         (((1,), (1,)), ((), ())),
                         preferred_element_type=jnp.float32)
    x = jnp.concatenate([numx_ref[...], e0, e1], axis=0)
    xw = lax.dot_general(x, W_ref[...],
                         (((1,), (0,)), ((), ())),
                         preferred_element_type=jnp.float32)
    deg = 1.0 + d0_ref[...] + d1_ref[...]
    dinv = lax.rsqrt(deg)
    dinv_ref[...] = dinv
    y_ref[...] = xw * dinv


def _tail_body(acc_ref, y_ref, dinv_ref, van_ref, fcW_ref, fcb_ref, out_ref):
    a = acc_ref[...]
    a0 = a[0:N]
    a1 = a[PADN:PADN + N]
    conv = jnp.maximum((a0 + a1 + y_ref[...]) * dinv_ref[...], 0.0)
    pooled = jnp.sum(conv, axis=0, keepdims=True) * (1.0 / N)
    fcW = fcW_ref[...]
    sc = lax.dot_general(pooled, fcW[NC:NC + NF],
                         (((1,), (0,)), ((), ())),
                         preferred_element_type=jnp.float32)
    z = lax.dot_general(van_ref[...], fcW[0:NC],
                        (((1,), (0,)), ((), ())),
                        preferred_element_type=jnp.float32)
    z = z + sc + fcb_ref[0, 0]
    beta = 1.1
    t = jax.nn.softplus(beta * z) / beta
    out_ref[...] = van_ref[...] / t


def kernel(num_x, cat_x0, cat_x1, edge_index, edge_weight, batch, vanilla_out,
           W_conv, embed_w0, embed_w1, fc_W, fc_b):
    npad = EPAD - E
    src2d = jnp.concatenate(
        [edge_index[0], jnp.zeros((npad,), jnp.int32)]).reshape(EPAD // CHUNK, CHUNK)
    dst2d = jnp.concatenate(
        [edge_index[1], jnp.zeros((npad,), jnp.int32)]).reshape(EPAD // CHUNK, CHUNK)
    ew2d = jnp.concatenate(
        [edge_weight, jnp.zeros((npad,), jnp.float32)]).reshape(EPAD // CHUNK, CHUNK)

    deg_flat = _deg_kernel(dst2d, ew2d)
    d0 = deg_flat[0:N].reshape(N, 1)
    d1 = deg_flat[PADN:PADN + N].reshape(N, 1)

    y, dinv = pl.pallas_call(
        _prep_body,
        out_shape=[
            jax.ShapeDtypeStruct((N, NF), jnp.float32),
            jax.ShapeDtypeStruct((N, 1), jnp.float32),
        ],
    )(num_x, cat_x0, cat_x1,
      embed_w0.reshape(1, 16), embed_w1.reshape(1, 16), W_conv, d0, d1)

    acc = _edge_kernel(y, src2d, dst2d, ew2d)

    out = pl.pallas_call(
        _tail_body,
        out_shape=jax.ShapeDtypeStruct((vanilla_out.shape[0], NC), jnp.float32),
    )(acc, y, dinv, vanilla_out, fc_W, fc_b.reshape(1, 1))
    return out
